# baseline (speedup 1.0000x reference)
import functools

import jax
import jax.numpy as jnp
from jax import lax
from jax.experimental import pallas as pl
from jax.experimental.pallas import tpu as pltpu
from jax.experimental.pallas import tpu_sc as plsc

_N = 10000
_E = 160000
_D = 256
_Q = 32
_NP = 10240
_RPT = _NP // 16
_NB_E = 80
_EPT = _NB_E * 128
_EPAD = 16 * _EPT
_BN = 512

_f32 = jnp.float32


def _mm_body(x_ref, w_ref, b_ref, o_ref):
    w = w_ref[0]
    o_ref[...] = (
        lax.dot_general(x_ref[...], w, (((1,), (1,)), ((), ())),
                        preferred_element_type=_f32)
        + b_ref[0, 0][None, :]
    )


_mm_call = pl.pallas_call(
    _mm_body,
    grid=(8, _NP // _BN),
    in_specs=[
        pl.BlockSpec((_BN, _D), lambda q, r: (r, jnp.int32(0))),
        pl.BlockSpec((1, _Q, _D), lambda q, r: (q, jnp.int32(0), jnp.int32(0))),
        pl.BlockSpec((1, 1, _Q), lambda q, r: (q, jnp.int32(0), jnp.int32(0))),
    ],
    out_specs=pl.BlockSpec((_BN, _Q), lambda q, r: (q * (_NP // _BN) + r, jnp.int32(0))),
    out_shape=jax.ShapeDtypeStruct((8 * _NP, _Q), _f32),
)


_mesh = plsc.VectorSubcoreMesh(core_axis_name="c", subcore_axis_name="s")


@functools.partial(
    pl.kernel,
    out_type=(
        jax.ShapeDtypeStruct((8, _NP, _Q), _f32),
        jax.ShapeDtypeStruct((8 * _NP, _Q), _f32),
        jax.ShapeDtypeStruct((8 * _NP, _Q), _f32),
    ),
    mesh=_mesh,
    compiler_params=pltpu.CompilerParams(needs_layout_passes=False,
                                         use_tc_tiling_on_sc=False),
    scratch_types=[
        pltpu.VMEM((_NB_E // 4, 512), jnp.int32),
        pltpu.VMEM((_NB_E // 4, 512), jnp.int32),
        pltpu.VMEM((_NB_E // 4, 512), jnp.int32),
        pltpu.VMEM((2, 512, _Q), _f32),
        pltpu.VMEM((128, _Q), _f32),
        pltpu.VMEM((128, _Q), _f32),
        pltpu.VMEM((128, _Q), _f32),
        pltpu.VMEM((128, _Q), _f32),
        pltpu.VMEM((_RPT, 16), _f32),
        pltpu.VMEM((_RPT, 16), _f32),
        pltpu.VMEM_SHARED((_NP, _Q), _f32),
        pltpu.SemaphoreType.DMA((2,)),
        pltpu.SemaphoreType.DMA,
    ],
)
def _sc_graph(h0, src4, dst4, hout, hs, hi,
              srcbuf, sidx, dstbuf, gbufs, gbuf, hsbuf, hibuf, zerobuf,
              n05buf, nl1buf, spacc, gsem, dsem):
    c = lax.axis_index("c")
    s = lax.axis_index("s")
    rbase = s * _RPT
    z16 = jnp.zeros((16,), _f32)
    one16 = jnp.ones((16,), _f32)

    pltpu.sync_copy(src4.at[s], srcbuf)
    pltpu.sync_copy(dst4.at[s], dstbuf)

    def zrow(i, carry):
        for m in range(_Q // 16):
            zerobuf[i, pl.ds(m * 16, 16)] = z16
        return carry
    lax.fori_loop(jnp.int32(0), jnp.int32(128), zrow, 0)

    def orow(i, carry):
        for m in range(_Q // 16):
            gbufs[0, i, pl.ds(m * 16, 16)] = one16
        return carry
    lax.fori_loop(jnp.int32(0), jnp.int32(512), orow, 0)

    def zacc(kk, carry):
        pltpu.sync_copy(zerobuf, spacc.at[pl.ds(rbase + kk * 128, 128)])
        return carry
    lax.fori_loop(jnp.int32(0), jnp.int32(_RPT // 128), zacc, 0)
    plsc.subcore_barrier()

    ones512 = gbufs.at[jnp.int32(0)]
    _sc1 = jax.named_scope("p1_deg"); _sc1.__enter__()

    def degb(i, carry):
        pltpu.async_copy(ones512, spacc.at[dstbuf.at[i]], dsem, add=True)
        return carry
    lax.fori_loop(jnp.int32(0), jnp.int32(_NB_E // 4), degb, 0)

    def degd(i, carry):
        pltpu.make_async_copy(ones512, spacc.at[dstbuf.at[i]], dsem).wait()
        return carry
    lax.fori_loop(jnp.int32(0), jnp.int32(_NB_E // 4), degd, 0)
    plsc.subcore_barrier()
    _sc1.__exit__(None, None, None)

    _sc2 = jax.named_scope("p2_norms"); _sc2.__enter__()
    def nchunk(kk, carry):
        pltpu.sync_copy(spacc.at[pl.ds(rbase + kk * 128, 128)], gbuf)

        def nrow(j, carry2):
            d = gbuf[j, pl.ds(0, 16)]
            x = jnp.maximum(d, 1.0)
            iv = plsc.bitcast(x, jnp.int32)
            iv = jnp.int32(0x5F3759DF) - lax.shift_right_arithmetic(iv, jnp.int32(1))
            y = plsc.bitcast(iv, _f32)
            for _ in range(3):
                y = y * (1.5 - 0.5 * x * y * y)
            n05buf[kk * 128 + j, :] = y
            nl1buf[kk * 128 + j, :] = y * y
            return carry2
        lax.fori_loop(jnp.int32(0), jnp.int32(128), nrow, 0)
        pltpu.sync_copy(zerobuf, spacc.at[pl.ds(rbase + kk * 128, 128)])
        return carry
    lax.fori_loop(jnp.int32(0), jnp.int32(_RPT // 128), nchunk, 0)
    _sc2.__exit__(None, None, None)

    _sc3 = jax.named_scope("p3_scale"); _sc3.__enter__()
    for q in range(4):
        qbase = (4 * c + q) * _NP + rbase
        def schunk(kk, carry, qbase=qbase):
            pltpu.sync_copy(h0.at[pl.ds(qbase + kk * 128, 128)], gbuf)

            def srow(j, carry2):
                nsp = n05buf[kk * 128 + j, :]
                lsp = nl1buf[kk * 128 + j, :]
                for m in range(_Q // 16):
                    v = gbuf[j, pl.ds(m * 16, 16)]
                    hsbuf[j, pl.ds(m * 16, 16)] = v * nsp
                    hibuf[j, pl.ds(m * 16, 16)] = v * lsp
                return carry2
            lax.fori_loop(jnp.int32(0), jnp.int32(128), srow, 0)
            pltpu.sync_copy(hsbuf, hs.at[pl.ds(qbase + kk * 128, 128)])
            pltpu.sync_copy(hibuf, hi.at[pl.ds(qbase + kk * 128, 128)])
            return carry
        lax.fori_loop(jnp.int32(0), jnp.int32(_RPT // 128), schunk, 0)
    plsc.subcore_barrier()
    _sc3.__exit__(None, None, None)

    for r in range(2):
        for q in range(4):
            qbase = (4 * c + q) * _NP + rbase
            qoffv = jnp.zeros((16,), jnp.int32) + (4 * c + q) * _NP

            def offrow(i, carry, qoffv=qoffv):
                for m in range(512 // 16):
                    sidx[i, pl.ds(m * 16, 16)] = (
                        srcbuf[i, pl.ds(m * 16, 16)] + qoffv)
                return carry
            lax.fori_loop(jnp.int32(0), jnp.int32(_NB_E // 4), offrow, 0)

            nsb = _NB_E // 4
            _sce = jax.named_scope("p4_edge_r%d_q%d" % (r, q)); _sce.__enter__()
            for p in range(2):
                pltpu.async_copy(hs.at[sidx.at[jnp.int32(p)]],
                                 gbufs.at[jnp.int32(p)], gsem.at[jnp.int32(p)])

            def edge(i, carry):
                p = lax.rem(i, jnp.int32(2))
                pltpu.make_async_copy(hs.at[sidx.at[i]], gbufs.at[p],
                                      gsem.at[p]).wait()
                pltpu.sync_copy(gbufs.at[p], spacc.at[dstbuf.at[i]], add=True)
                pltpu.async_copy(hs.at[sidx.at[i + 2]], gbufs.at[p], gsem.at[p])
                return carry
            lax.fori_loop(jnp.int32(0), jnp.int32(nsb - 2), edge, 0)
            for t in (nsb - 2, nsb - 1):
                tt, pp = jnp.int32(t), jnp.int32(t % 2)
                pltpu.make_async_copy(hs.at[sidx.at[tt]], gbufs.at[pp],
                                      gsem.at[pp]).wait()
                pltpu.sync_copy(gbufs.at[pp], spacc.at[dstbuf.at[tt]], add=True)
            plsc.subcore_barrier()
            _sce.__exit__(None, None, None)
            _scp = jax.named_scope("p5_epi_r%d_q%d" % (r, q)); _scp.__enter__()

            def echunk(kk, carry, qbase=qbase, r=r, q=q):
                pltpu.sync_copy(spacc.at[pl.ds(rbase + kk * 128, 128)], gbuf)
                pltpu.sync_copy(hi.at[pl.ds(qbase + kk * 128, 128)], hibuf)

                def erow(j, carry2):
                    nsp = n05buf[kk * 128 + j, :]
                    for m in range(_Q // 16):
                        a = gbuf[j, pl.ds(m * 16, 16)]
                        hnew = a * nsp + hibuf[j, pl.ds(m * 16, 16)]
                        if r == 0:
                            hsbuf[j, pl.ds(m * 16, 16)] = hnew * nsp
                        else:
                            hsbuf[j, pl.ds(m * 16, 16)] = hnew
                    return carry2
                lax.fori_loop(jnp.int32(0), jnp.int32(128), erow, 0)
                pltpu.sync_copy(zerobuf, spacc.at[pl.ds(rbase + kk * 128, 128)])
                if r == 0:
                    pltpu.sync_copy(hsbuf, hs.at[pl.ds(qbase + kk * 128, 128)])
                else:
                    pltpu.sync_copy(hsbuf,
                                    hout.at[4 * c + jnp.int32(q), pl.ds(rbase + kk * 128, 128)])
                return carry
            lax.fori_loop(jnp.int32(0), jnp.int32(_RPT // 128), echunk, 0)
            plsc.subcore_barrier()
            _scp.__exit__(None, None, None)


def kernel(features, edge_index, W, b):
    src = edge_index[0].astype(jnp.int32)
    dst = edge_index[1].astype(jnp.int32)
    pad = _EPAD - _E
    srcp = jnp.concatenate([src, jnp.zeros((pad,), jnp.int32)])
    dstp = jnp.concatenate([dst, jnp.full((pad,), _N, jnp.int32)])
    src4 = srcp.reshape(16, _NB_E // 4, 512)
    dst4 = dstp.reshape(16, _NB_E // 4, 512)
    feats_p = jnp.pad(features, ((0, _NP - _N), (0, 0)))
    Wr = W.reshape(8, _Q, _D)
    br = b.reshape(8, 1, _Q)
    h0 = _mm_call(feats_p, Wr, br)
    hout, _, _ = _sc_graph(h0, src4, dst4)
    return hout.transpose(1, 0, 2).reshape(_NP, _D)[:_N]


# speedup vs baseline: 1.0817x; 1.0817x over previous
import functools

import jax
import jax.numpy as jnp
from jax import lax
from jax.experimental import pallas as pl
from jax.experimental.pallas import tpu as pltpu
from jax.experimental.pallas import tpu_sc as plsc

_N = 10000
_E = 160000
_D = 256
_Q = 32
_NP = 10240
_RPT = _NP // 16
_NB_E = 80
_EPT = _NB_E * 128
_EPAD = 16 * _EPT
_BN = 512

_f32 = jnp.float32


def _mm_body(x_ref, w_ref, b_ref, o_ref):
    w = w_ref[0]
    o_ref[...] = (
        lax.dot_general(x_ref[...], w, (((1,), (1,)), ((), ())),
                        preferred_element_type=_f32)
        + b_ref[0, 0][None, :]
    )


_mm_call = pl.pallas_call(
    _mm_body,
    grid=(8, _NP // _BN),
    in_specs=[
        pl.BlockSpec((_BN, _D), lambda q, r: (r, jnp.int32(0))),
        pl.BlockSpec((1, _Q, _D), lambda q, r: (q, jnp.int32(0), jnp.int32(0))),
        pl.BlockSpec((1, 1, _Q), lambda q, r: (q, jnp.int32(0), jnp.int32(0))),
    ],
    out_specs=pl.BlockSpec((_BN, _Q), lambda q, r: (q * (_NP // _BN) + r, jnp.int32(0))),
    out_shape=jax.ShapeDtypeStruct((8 * _NP, _Q), _f32),
)


_mesh = plsc.VectorSubcoreMesh(core_axis_name="c", subcore_axis_name="s")


@functools.partial(
    pl.kernel,
    out_type=(
        jax.ShapeDtypeStruct((_NP, _D), _f32),
        jax.ShapeDtypeStruct((8 * _NP, _Q), _f32),
        jax.ShapeDtypeStruct((8 * _NP, _Q), _f32),
    ),
    mesh=_mesh,
    compiler_params=pltpu.CompilerParams(needs_layout_passes=False,
                                         use_tc_tiling_on_sc=False),
    scratch_types=[
        pltpu.VMEM((_NB_E // 4, 512), jnp.int32),
        pltpu.VMEM((_NB_E // 4, 512), jnp.int32),
        pltpu.VMEM((_NB_E // 4, 512), jnp.int32),
        pltpu.VMEM((2, 512, _Q), _f32),
        pltpu.VMEM((128, _Q), _f32),
        pltpu.VMEM((128, _Q), _f32),
        pltpu.VMEM((128, _Q), _f32),
        pltpu.VMEM((128, _Q), _f32),
        pltpu.VMEM((_RPT, 16), _f32),
        pltpu.VMEM((_RPT, 16), _f32),
        pltpu.VMEM_SHARED((_NP, _Q), _f32),
        pltpu.SemaphoreType.DMA((2,)),
        pltpu.SemaphoreType.DMA,
    ],
)
def _sc_graph(h0, src4, dst4, hout, hs, hi,
              srcbuf, sidx, dstbuf, gbufs, gbuf, hsbuf, hibuf, zerobuf,
              n05buf, nl1buf, spacc, gsem, dsem):
    c = lax.axis_index("c")
    s = lax.axis_index("s")
    rbase = s * _RPT
    z16 = jnp.zeros((16,), _f32)
    one16 = jnp.ones((16,), _f32)

    pltpu.sync_copy(src4.at[s], srcbuf)
    pltpu.sync_copy(dst4.at[s], dstbuf)

    def zrow(i, carry):
        for m in range(_Q // 16):
            zerobuf[i, pl.ds(m * 16, 16)] = z16
        return carry
    lax.fori_loop(jnp.int32(0), jnp.int32(128), zrow, 0)

    def orow(i, carry):
        for m in range(_Q // 16):
            gbufs[0, i, pl.ds(m * 16, 16)] = one16
        return carry
    lax.fori_loop(jnp.int32(0), jnp.int32(512), orow, 0)

    def zacc(kk, carry):
        pltpu.sync_copy(zerobuf, spacc.at[pl.ds(rbase + kk * 128, 128)])
        return carry
    lax.fori_loop(jnp.int32(0), jnp.int32(_RPT // 128), zacc, 0)
    plsc.subcore_barrier()

    ones512 = gbufs.at[jnp.int32(0)]
    _sc1 = jax.named_scope("p1_deg"); _sc1.__enter__()

    def degb(i, carry):
        pltpu.async_copy(ones512, spacc.at[dstbuf.at[i]], dsem, add=True)
        return carry
    lax.fori_loop(jnp.int32(0), jnp.int32(_NB_E // 4), degb, 0)

    def degd(i, carry):
        pltpu.make_async_copy(ones512, spacc.at[dstbuf.at[i]], dsem).wait()
        return carry
    lax.fori_loop(jnp.int32(0), jnp.int32(_NB_E // 4), degd, 0)
    plsc.subcore_barrier()
    _sc1.__exit__(None, None, None)

    _sc2 = jax.named_scope("p2_norms"); _sc2.__enter__()
    def nchunk(kk, carry):
        pltpu.sync_copy(spacc.at[pl.ds(rbase + kk * 128, 128)], gbuf)

        def nrow(j, carry2):
            d = gbuf[j, pl.ds(0, 16)]
            x = jnp.maximum(d, 1.0)
            iv = plsc.bitcast(x, jnp.int32)
            iv = jnp.int32(0x5F3759DF) - lax.shift_right_arithmetic(iv, jnp.int32(1))
            y = plsc.bitcast(iv, _f32)
            for _ in range(3):
                y = y * (1.5 - 0.5 * x * y * y)
            n05buf[kk * 128 + j, :] = y
            nl1buf[kk * 128 + j, :] = y * y
            return carry2
        lax.fori_loop(jnp.int32(0), jnp.int32(128), nrow, 0)
        pltpu.sync_copy(zerobuf, spacc.at[pl.ds(rbase + kk * 128, 128)])
        return carry
    lax.fori_loop(jnp.int32(0), jnp.int32(_RPT // 128), nchunk, 0)
    _sc2.__exit__(None, None, None)

    _sc3 = jax.named_scope("p3_scale"); _sc3.__enter__()
    for q in range(4):
        qbase = (4 * c + q) * _NP + rbase
        def schunk(kk, carry, qbase=qbase):
            pltpu.sync_copy(h0.at[pl.ds(qbase + kk * 128, 128)], gbuf)

            def srow(j, carry2):
                nsp = n05buf[kk * 128 + j, :]
                lsp = nl1buf[kk * 128 + j, :]
                for m in range(_Q // 16):
                    v = gbuf[j, pl.ds(m * 16, 16)]
                    hsbuf[j, pl.ds(m * 16, 16)] = v * nsp
                    hibuf[j, pl.ds(m * 16, 16)] = v * lsp
                return carry2
            lax.fori_loop(jnp.int32(0), jnp.int32(128), srow, 0)
            pltpu.sync_copy(hsbuf, hs.at[pl.ds(qbase + kk * 128, 128)])
            pltpu.sync_copy(hibuf, hi.at[pl.ds(qbase + kk * 128, 128)])
            return carry
        lax.fori_loop(jnp.int32(0), jnp.int32(_RPT // 128), schunk, 0)
    plsc.subcore_barrier()
    _sc3.__exit__(None, None, None)

    for r in range(2):
        for q in range(4):
            qbase = (4 * c + q) * _NP + rbase
            qoffv = jnp.zeros((16,), jnp.int32) + (4 * c + q) * _NP

            def offrow(i, carry, qoffv=qoffv):
                for m in range(512 // 16):
                    sidx[i, pl.ds(m * 16, 16)] = (
                        srcbuf[i, pl.ds(m * 16, 16)] + qoffv)
                return carry
            lax.fori_loop(jnp.int32(0), jnp.int32(_NB_E // 4), offrow, 0)

            nsb = _NB_E // 4
            _sce = jax.named_scope("p4_edge_r%d_q%d" % (r, q)); _sce.__enter__()
            for p in range(2):
                pltpu.async_copy(hs.at[sidx.at[jnp.int32(p)]],
                                 gbufs.at[jnp.int32(p)], gsem.at[jnp.int32(p)])

            def edge(i, carry):
                p = lax.rem(i, jnp.int32(2))
                pltpu.make_async_copy(hs.at[sidx.at[i]], gbufs.at[p],
                                      gsem.at[p]).wait()
                pltpu.sync_copy(gbufs.at[p], spacc.at[dstbuf.at[i]], add=True)
                pltpu.async_copy(hs.at[sidx.at[i + 2]], gbufs.at[p], gsem.at[p])
                return carry
            lax.fori_loop(jnp.int32(0), jnp.int32(nsb - 2), edge, 0)
            for t in (nsb - 2, nsb - 1):
                tt, pp = jnp.int32(t), jnp.int32(t % 2)
                pltpu.make_async_copy(hs.at[sidx.at[tt]], gbufs.at[pp],
                                      gsem.at[pp]).wait()
                pltpu.sync_copy(gbufs.at[pp], spacc.at[dstbuf.at[tt]], add=True)
            plsc.subcore_barrier()
            _sce.__exit__(None, None, None)
            _scp = jax.named_scope("p5_epi_r%d_q%d" % (r, q)); _scp.__enter__()

            def echunk(kk, carry, qbase=qbase, r=r, q=q):
                pltpu.sync_copy(spacc.at[pl.ds(rbase + kk * 128, 128)], gbuf)
                pltpu.sync_copy(hi.at[pl.ds(qbase + kk * 128, 128)], hibuf)

                def erow(j, carry2):
                    nsp = n05buf[kk * 128 + j, :]
                    for m in range(_Q // 16):
                        a = gbuf[j, pl.ds(m * 16, 16)]
                        hnew = a * nsp + hibuf[j, pl.ds(m * 16, 16)]
                        if r == 0:
                            hsbuf[j, pl.ds(m * 16, 16)] = hnew * nsp
                        else:
                            hsbuf[j, pl.ds(m * 16, 16)] = hnew
                    return carry2
                lax.fori_loop(jnp.int32(0), jnp.int32(128), erow, 0)
                pltpu.sync_copy(zerobuf, spacc.at[pl.ds(rbase + kk * 128, 128)])
                if r == 0:
                    pltpu.sync_copy(hsbuf, hs.at[pl.ds(qbase + kk * 128, 128)])
                else:
                    pltpu.sync_copy(hsbuf,
                                    hout.at[pl.ds(rbase + kk * 128, 128),
                                            pl.ds((4 * c + jnp.int32(q)) * _Q, _Q)])
                return carry
            lax.fori_loop(jnp.int32(0), jnp.int32(_RPT // 128), echunk, 0)
            plsc.subcore_barrier()
            _scp.__exit__(None, None, None)


def kernel(features, edge_index, W, b):
    src = edge_index[0].astype(jnp.int32)
    dst = edge_index[1].astype(jnp.int32)
    pad = _EPAD - _E
    srcp = jnp.concatenate([src, jnp.zeros((pad,), jnp.int32)])
    dstp = jnp.concatenate([dst, jnp.full((pad,), _N, jnp.int32)])
    src4 = srcp.reshape(16, _NB_E // 4, 512)
    dst4 = dstp.reshape(16, _NB_E // 4, 512)
    feats_p = jnp.pad(features, ((0, _NP - _N), (0, 0)))
    Wr = W.reshape(8, _Q, _D)
    br = b.reshape(8, 1, _Q)
    h0 = _mm_call(feats_p, Wr, br)
    hout, _, _ = _sc_graph(h0, src4, dst4)
    return hout[:_N]


# speedup vs baseline: 1.2865x; 1.1894x over previous
import functools

import jax
import jax.numpy as jnp
from jax import lax
from jax.experimental import pallas as pl
from jax.experimental.pallas import tpu as pltpu
from jax.experimental.pallas import tpu_sc as plsc

_N = 10000
_E = 160000
_D = 256
_Q = 32
_NP = 10240
_RPT = _NP // 16
_NB_E = 80
_EPT = _NB_E * 128
_EPAD = 16 * _EPT
_BN = 512

_f32 = jnp.float32


def _mm_body(x_ref, w_ref, b_ref, o_ref):
    o_ref[...] = (
        lax.dot_general(x_ref[...], w_ref[...], (((1,), (1,)), ((), ())),
                        preferred_element_type=_f32)
        + b_ref[...]
    )


_mm_call = pl.pallas_call(
    _mm_body,
    grid=(_NP // _BN,),
    in_specs=[
        pl.BlockSpec((_BN, _D), lambda r: (r, jnp.int32(0))),
        pl.BlockSpec((_D, _D), lambda r: (jnp.int32(0), jnp.int32(0))),
        pl.BlockSpec((1, _D), lambda r: (jnp.int32(0), jnp.int32(0))),
    ],
    out_specs=pl.BlockSpec((_BN, _D), lambda r: (r, jnp.int32(0))),
    out_shape=jax.ShapeDtypeStruct((_NP, _D), _f32),
)


_mesh = plsc.VectorSubcoreMesh(core_axis_name="c", subcore_axis_name="s")


@functools.partial(
    pl.kernel,
    out_type=(
        jax.ShapeDtypeStruct((_NP, _D), _f32),
        jax.ShapeDtypeStruct((8 * _NP, _Q), _f32),
        jax.ShapeDtypeStruct((8 * _NP, _Q), _f32),
    ),
    mesh=_mesh,
    compiler_params=pltpu.CompilerParams(needs_layout_passes=False,
                                         use_tc_tiling_on_sc=False),
    scratch_types=[
        pltpu.VMEM((_NB_E // 4, 512), jnp.int32),
        pltpu.VMEM((_NB_E // 4, 512), jnp.int32),
        pltpu.VMEM((_NB_E // 4, 512), jnp.int32),
        pltpu.VMEM((2, 512, _Q), _f32),
        pltpu.VMEM((128, _Q), _f32),
        pltpu.VMEM((128, _Q), _f32),
        pltpu.VMEM((128, _Q), _f32),
        pltpu.VMEM((128, _Q), _f32),
        pltpu.VMEM((_RPT, 16), _f32),
        pltpu.VMEM((_RPT, 16), _f32),
        pltpu.VMEM_SHARED((_NP, _Q), _f32),
        pltpu.SemaphoreType.DMA((2,)),
        pltpu.SemaphoreType.DMA,
    ],
)
def _sc_graph(h0, src4, dst4, hout, hs, hi,
              srcbuf, sidx, dstbuf, gbufs, gbuf, hsbuf, hibuf, zerobuf,
              n05buf, nl1buf, spacc, gsem, dsem):
    c = lax.axis_index("c")
    s = lax.axis_index("s")
    rbase = s * _RPT
    z16 = jnp.zeros((16,), _f32)
    one16 = jnp.ones((16,), _f32)

    pltpu.sync_copy(src4.at[s], srcbuf)
    pltpu.sync_copy(dst4.at[s], dstbuf)

    def zrow(i, carry):
        for m in range(_Q // 16):
            zerobuf[i, pl.ds(m * 16, 16)] = z16
        return carry
    lax.fori_loop(jnp.int32(0), jnp.int32(128), zrow, 0)

    def orow(i, carry):
        for m in range(_Q // 16):
            gbufs[0, i, pl.ds(m * 16, 16)] = one16
        return carry
    lax.fori_loop(jnp.int32(0), jnp.int32(512), orow, 0)

    def zacc(kk, carry):
        pltpu.sync_copy(zerobuf, spacc.at[pl.ds(rbase + kk * 128, 128)])
        return carry
    lax.fori_loop(jnp.int32(0), jnp.int32(_RPT // 128), zacc, 0)
    plsc.subcore_barrier()

    ones512 = gbufs.at[jnp.int32(0)]
    _sc1 = jax.named_scope("p1_deg"); _sc1.__enter__()

    def degb(i, carry):
        pltpu.async_copy(ones512, spacc.at[dstbuf.at[i]], dsem, add=True)
        return carry
    lax.fori_loop(jnp.int32(0), jnp.int32(_NB_E // 4), degb, 0)

    def degd(i, carry):
        pltpu.make_async_copy(ones512, spacc.at[dstbuf.at[i]], dsem).wait()
        return carry
    lax.fori_loop(jnp.int32(0), jnp.int32(_NB_E // 4), degd, 0)
    plsc.subcore_barrier()
    _sc1.__exit__(None, None, None)

    _sc2 = jax.named_scope("p2_norms"); _sc2.__enter__()
    def nchunk(kk, carry):
        pltpu.sync_copy(spacc.at[pl.ds(rbase + kk * 128, 128)], gbuf)

        def nrow(j, carry2):
            d = gbuf[j, pl.ds(0, 16)]
            x = jnp.maximum(d, 1.0)
            iv = plsc.bitcast(x, jnp.int32)
            iv = jnp.int32(0x5F3759DF) - lax.shift_right_arithmetic(iv, jnp.int32(1))
            y = plsc.bitcast(iv, _f32)
            for _ in range(3):
                y = y * (1.5 - 0.5 * x * y * y)
            n05buf[kk * 128 + j, :] = y
            nl1buf[kk * 128 + j, :] = y * y
            return carry2
        lax.fori_loop(jnp.int32(0), jnp.int32(128), nrow, 0)
        pltpu.sync_copy(zerobuf, spacc.at[pl.ds(rbase + kk * 128, 128)])
        return carry
    lax.fori_loop(jnp.int32(0), jnp.int32(_RPT // 128), nchunk, 0)
    _sc2.__exit__(None, None, None)

    _sc3 = jax.named_scope("p3_scale"); _sc3.__enter__()
    for q in range(4):
        qbase = (4 * c + q) * _NP + rbase
        def schunk(kk, carry, qbase=qbase):
            pltpu.sync_copy(h0.at[pl.ds(rbase + kk * 128, 128),
                                  pl.ds((4 * c + jnp.int32(q)) * _Q, _Q)], gbuf)

            def srow(j, carry2):
                nsp = n05buf[kk * 128 + j, :]
                lsp = nl1buf[kk * 128 + j, :]
                for m in range(_Q // 16):
                    v = gbuf[j, pl.ds(m * 16, 16)]
                    hsbuf[j, pl.ds(m * 16, 16)] = v * nsp
                    hibuf[j, pl.ds(m * 16, 16)] = v * lsp
                return carry2
            lax.fori_loop(jnp.int32(0), jnp.int32(128), srow, 0)
            pltpu.sync_copy(hsbuf, hs.at[pl.ds(qbase + kk * 128, 128)])
            pltpu.sync_copy(hibuf, hi.at[pl.ds(qbase + kk * 128, 128)])
            return carry
        lax.fori_loop(jnp.int32(0), jnp.int32(_RPT // 128), schunk, 0)
    plsc.subcore_barrier()
    _sc3.__exit__(None, None, None)

    for r in range(2):
        for q in range(4):
            qbase = (4 * c + q) * _NP + rbase
            qoffv = jnp.zeros((16,), jnp.int32) + (4 * c + q) * _NP

            def offrow(i, carry, qoffv=qoffv):
                for m in range(512 // 16):
                    sidx[i, pl.ds(m * 16, 16)] = (
                        srcbuf[i, pl.ds(m * 16, 16)] + qoffv)
                return carry
            lax.fori_loop(jnp.int32(0), jnp.int32(_NB_E // 4), offrow, 0)

            nsb = _NB_E // 4
            _sce = jax.named_scope("p4_edge_r%d_q%d" % (r, q)); _sce.__enter__()
            for p in range(2):
                pltpu.async_copy(hs.at[sidx.at[jnp.int32(p)]],
                                 gbufs.at[jnp.int32(p)], gsem.at[jnp.int32(p)])

            def edge(i, carry):
                p = lax.rem(i, jnp.int32(2))
                pltpu.make_async_copy(hs.at[sidx.at[i]], gbufs.at[p],
                                      gsem.at[p]).wait()
                pltpu.sync_copy(gbufs.at[p], spacc.at[dstbuf.at[i]], add=True)
                pltpu.async_copy(hs.at[sidx.at[i + 2]], gbufs.at[p], gsem.at[p])
                return carry
            lax.fori_loop(jnp.int32(0), jnp.int32(nsb - 2), edge, 0)
            for t in (nsb - 2, nsb - 1):
                tt, pp = jnp.int32(t), jnp.int32(t % 2)
                pltpu.make_async_copy(hs.at[sidx.at[tt]], gbufs.at[pp],
                                      gsem.at[pp]).wait()
                pltpu.sync_copy(gbufs.at[pp], spacc.at[dstbuf.at[tt]], add=True)
            plsc.subcore_barrier()
            _sce.__exit__(None, None, None)
            _scp = jax.named_scope("p5_epi_r%d_q%d" % (r, q)); _scp.__enter__()

            def echunk(kk, carry, qbase=qbase, r=r, q=q):
                pltpu.sync_copy(spacc.at[pl.ds(rbase + kk * 128, 128)], gbuf)
                pltpu.sync_copy(hi.at[pl.ds(qbase + kk * 128, 128)], hibuf)

                def erow(j, carry2):
                    nsp = n05buf[kk * 128 + j, :]
                    for m in range(_Q // 16):
                        a = gbuf[j, pl.ds(m * 16, 16)]
                        hnew = a * nsp + hibuf[j, pl.ds(m * 16, 16)]
                        if r == 0:
                            hsbuf[j, pl.ds(m * 16, 16)] = hnew * nsp
                        else:
                            hsbuf[j, pl.ds(m * 16, 16)] = hnew
                    return carry2
                lax.fori_loop(jnp.int32(0), jnp.int32(128), erow, 0)
                pltpu.sync_copy(zerobuf, spacc.at[pl.ds(rbase + kk * 128, 128)])
                if r == 0:
                    pltpu.sync_copy(hsbuf, hs.at[pl.ds(qbase + kk * 128, 128)])
                else:
                    pltpu.sync_copy(hsbuf,
                                    hout.at[pl.ds(rbase + kk * 128, 128),
                                            pl.ds((4 * c + jnp.int32(q)) * _Q, _Q)])
                return carry
            lax.fori_loop(jnp.int32(0), jnp.int32(_RPT // 128), echunk, 0)
            plsc.subcore_barrier()
            _scp.__exit__(None, None, None)


def kernel(features, edge_index, W, b):
    src = edge_index[0].astype(jnp.int32)
    dst = edge_index[1].astype(jnp.int32)
    pad = _EPAD - _E
    srcp = jnp.concatenate([src, jnp.zeros((pad,), jnp.int32)])
    dstp = jnp.concatenate([dst, jnp.full((pad,), _N, jnp.int32)])
    src4 = srcp.reshape(16, _NB_E // 4, 512)
    dst4 = dstp.reshape(16, _NB_E // 4, 512)
    feats_p = jnp.pad(features, ((0, _NP - _N), (0, 0)))
    h0 = _mm_call(feats_p, W, b.reshape(1, _D))
    hout, _, _ = _sc_graph(h0, src4, dst4)
    return hout[:_N]


# speedup vs baseline: 1.2879x; 1.0011x over previous
import functools

import jax
import jax.numpy as jnp
from jax import lax
from jax.experimental import pallas as pl
from jax.experimental.pallas import tpu as pltpu
from jax.experimental.pallas import tpu_sc as plsc

_N = 10000
_E = 160000
_D = 256
_Q = 32
_NP = 10240
_RPT = _NP // 16
_NB_E = 80
_EPT = _NB_E * 128
_EPAD = 16 * _EPT
_BN = 512

_f32 = jnp.float32


def _mm_body(x_ref, w_ref, b_ref, o_ref):
    o_ref[...] = (
        lax.dot_general(x_ref[...], w_ref[...], (((1,), (1,)), ((), ())),
                        preferred_element_type=_f32)
        + b_ref[...]
    )


_mm_call = pl.pallas_call(
    _mm_body,
    grid=(_NP // _BN,),
    in_specs=[
        pl.BlockSpec((_BN, _D), lambda r: (r, jnp.int32(0))),
        pl.BlockSpec((_D, _D), lambda r: (jnp.int32(0), jnp.int32(0))),
        pl.BlockSpec((1, _D), lambda r: (jnp.int32(0), jnp.int32(0))),
    ],
    out_specs=pl.BlockSpec((_BN, _D), lambda r: (r, jnp.int32(0))),
    out_shape=jax.ShapeDtypeStruct((_NP, _D), _f32),
)


_mesh = plsc.VectorSubcoreMesh(core_axis_name="c", subcore_axis_name="s")


@functools.partial(
    pl.kernel,
    out_type=(
        jax.ShapeDtypeStruct((_NP, _D), _f32),
        jax.ShapeDtypeStruct((8 * _NP, _Q), _f32),
        jax.ShapeDtypeStruct((8 * _NP, _Q), _f32),
    ),
    mesh=_mesh,
    compiler_params=pltpu.CompilerParams(needs_layout_passes=False,
                                         use_tc_tiling_on_sc=False),
    scratch_types=[
        pltpu.VMEM((_NB_E // 4, 512), jnp.int32),
        pltpu.VMEM((_NB_E // 4, 512), jnp.int32),
        pltpu.VMEM((_NB_E // 4, 512), jnp.int32),
        pltpu.VMEM((2, 512, _Q), _f32),
        pltpu.VMEM((128, _Q), _f32),
        pltpu.VMEM((128, _Q), _f32),
        pltpu.VMEM((128, _Q), _f32),
        pltpu.VMEM((128, _Q), _f32),
        pltpu.VMEM((_RPT, 16), _f32),
        pltpu.VMEM((_RPT, 16), _f32),
        pltpu.VMEM_SHARED((_NP, _Q), _f32),
        pltpu.SemaphoreType.DMA((2,)),
        pltpu.SemaphoreType.DMA,
    ],
)
def _sc_graph(h0, src4, dst4, hout, hs, hi,
              srcbuf, sidx, dstbuf, gbufs, gbuf, hsbuf, hibuf, zerobuf,
              n05buf, nl1buf, spacc, gsem, dsem):
    c = lax.axis_index("c")
    s = lax.axis_index("s")
    rbase = s * _RPT
    z16 = jnp.zeros((16,), _f32)
    one16 = jnp.ones((16,), _f32)

    pltpu.sync_copy(src4.at[s], srcbuf)
    pltpu.sync_copy(dst4.at[s], dstbuf)

    def zrow(i, carry):
        for m in range(_Q // 16):
            zerobuf[i, pl.ds(m * 16, 16)] = z16
        return carry
    lax.fori_loop(jnp.int32(0), jnp.int32(128), zrow, 0)

    def orow(i, carry):
        for m in range(_Q // 16):
            gbufs[0, i, pl.ds(m * 16, 16)] = one16
        return carry
    lax.fori_loop(jnp.int32(0), jnp.int32(512), orow, 0)

    def zacc(kk, carry):
        pltpu.sync_copy(zerobuf, spacc.at[pl.ds(rbase + kk * 128, 128)])
        return carry
    lax.fori_loop(jnp.int32(0), jnp.int32(_RPT // 128), zacc, 0)
    plsc.subcore_barrier()

    ones512 = gbufs.at[jnp.int32(0)]
    _sc1 = jax.named_scope("p1_deg"); _sc1.__enter__()

    def degb(i, carry):
        pltpu.async_copy(ones512, spacc.at[dstbuf.at[i]], dsem, add=True)
        return carry
    lax.fori_loop(jnp.int32(0), jnp.int32(_NB_E // 4), degb, 0)

    def degd(i, carry):
        pltpu.make_async_copy(ones512, spacc.at[dstbuf.at[i]], dsem).wait()
        return carry
    lax.fori_loop(jnp.int32(0), jnp.int32(_NB_E // 4), degd, 0)
    plsc.subcore_barrier()
    _sc1.__exit__(None, None, None)

    _sc2 = jax.named_scope("p2_norms"); _sc2.__enter__()
    def nchunk(kk, carry):
        pltpu.sync_copy(spacc.at[pl.ds(rbase + kk * 128, 128)], gbuf)

        def nrow(j, carry2):
            d = gbuf[j, pl.ds(0, 16)]
            x = jnp.maximum(d, 1.0)
            iv = plsc.bitcast(x, jnp.int32)
            iv = jnp.int32(0x5F3759DF) - lax.shift_right_arithmetic(iv, jnp.int32(1))
            y = plsc.bitcast(iv, _f32)
            for _ in range(3):
                y = y * (1.5 - 0.5 * x * y * y)
            n05buf[kk * 128 + j, :] = y
            nl1buf[kk * 128 + j, :] = y * y
            return carry2
        lax.fori_loop(jnp.int32(0), jnp.int32(128), nrow, 0)
        pltpu.sync_copy(zerobuf, spacc.at[pl.ds(rbase + kk * 128, 128)])
        return carry
    lax.fori_loop(jnp.int32(0), jnp.int32(_RPT // 128), nchunk, 0)
    _sc2.__exit__(None, None, None)

    _sc3 = jax.named_scope("p3_scale"); _sc3.__enter__()
    for q in range(4):
        qbase = (4 * c + q) * _NP + rbase
        def schunk(kk, carry, qbase=qbase):
            pltpu.sync_copy(h0.at[pl.ds(rbase + kk * 128, 128),
                                  pl.ds((4 * c + jnp.int32(q)) * _Q, _Q)], gbuf)

            def srow(j, carry2):
                nsp = n05buf[kk * 128 + j, :]
                lsp = nl1buf[kk * 128 + j, :]
                for m in range(_Q // 16):
                    v = gbuf[j, pl.ds(m * 16, 16)]
                    hsbuf[j, pl.ds(m * 16, 16)] = v * nsp
                    hibuf[j, pl.ds(m * 16, 16)] = v * lsp
                return carry2
            lax.fori_loop(jnp.int32(0), jnp.int32(128), srow, 0)
            pltpu.sync_copy(hsbuf, hs.at[pl.ds(qbase + kk * 128, 128)])
            pltpu.sync_copy(hibuf, hi.at[pl.ds(qbase + kk * 128, 128)])
            return carry
        lax.fori_loop(jnp.int32(0), jnp.int32(_RPT // 128), schunk, 0)
    plsc.subcore_barrier()
    _sc3.__exit__(None, None, None)

    for r in range(2):
        for q in range(4):
            qbase = (4 * c + q) * _NP + rbase
            qoffv = jnp.zeros((16,), jnp.int32) + (4 * c + q) * _NP

            def offrow(i, carry, qoffv=qoffv):
                for m in range(512 // 16):
                    sidx[i, pl.ds(m * 16, 16)] = (
                        srcbuf[i, pl.ds(m * 16, 16)] + qoffv)
                return carry
            lax.fori_loop(jnp.int32(0), jnp.int32(_NB_E // 4), offrow, 0)

            nsb = _NB_E // 4
            _sce = jax.named_scope("p4_edge_r%d_q%d" % (r, q)); _sce.__enter__()
            for p in range(2):
                pltpu.async_copy(hs.at[sidx.at[jnp.int32(p)]],
                                 gbufs.at[jnp.int32(p)], gsem.at[jnp.int32(p)])

            def edge(i, carry):
                p = lax.rem(i, jnp.int32(2))
                pltpu.make_async_copy(hs.at[sidx.at[i]], gbufs.at[p],
                                      gsem.at[p]).wait()
                pltpu.sync_copy(gbufs.at[p], spacc.at[dstbuf.at[i]], add=True)
                pltpu.async_copy(hs.at[sidx.at[i + 2]], gbufs.at[p], gsem.at[p])
                return carry
            lax.fori_loop(jnp.int32(0), jnp.int32(nsb - 2), edge, 0)
            for t in (nsb - 2, nsb - 1):
                tt, pp = jnp.int32(t), jnp.int32(t % 2)
                pltpu.make_async_copy(hs.at[sidx.at[tt]], gbufs.at[pp],
                                      gsem.at[pp]).wait()
                pltpu.sync_copy(gbufs.at[pp], spacc.at[dstbuf.at[tt]], add=True)
            plsc.subcore_barrier()
            _sce.__exit__(None, None, None)
            _scp = jax.named_scope("p5_epi_r%d_q%d" % (r, q)); _scp.__enter__()

            def echunk(kk, carry, qbase=qbase, r=r, q=q):
                pltpu.sync_copy(spacc.at[pl.ds(rbase + kk * 128, 128)], gbuf)
                pltpu.sync_copy(hi.at[pl.ds(qbase + kk * 128, 128)], hibuf)

                def erow(j, carry2):
                    nsp = n05buf[kk * 128 + j, :]
                    for m in range(_Q // 16):
                        a = gbuf[j, pl.ds(m * 16, 16)]
                        hnew = a * nsp + hibuf[j, pl.ds(m * 16, 16)]
                        if r == 0:
                            hsbuf[j, pl.ds(m * 16, 16)] = hnew * nsp
                        else:
                            hsbuf[j, pl.ds(m * 16, 16)] = hnew
                    return carry2
                lax.fori_loop(jnp.int32(0), jnp.int32(128), erow, 0)
                if not (r == 1 and q == 3):
                    pltpu.sync_copy(zerobuf,
                                    spacc.at[pl.ds(rbase + kk * 128, 128)])
                if r == 0:
                    pltpu.sync_copy(hsbuf, hs.at[pl.ds(qbase + kk * 128, 128)])
                else:
                    pltpu.sync_copy(hsbuf,
                                    hout.at[pl.ds(rbase + kk * 128, 128),
                                            pl.ds((4 * c + jnp.int32(q)) * _Q, _Q)])
                return carry
            lax.fori_loop(jnp.int32(0), jnp.int32(_RPT // 128), echunk, 0)
            plsc.subcore_barrier()
            _scp.__exit__(None, None, None)


def kernel(features, edge_index, W, b):
    src = edge_index[0].astype(jnp.int32)
    dst = edge_index[1].astype(jnp.int32)
    pad = _EPAD - _E
    srcp = jnp.concatenate([src, jnp.zeros((pad,), jnp.int32)])
    dstp = jnp.concatenate([dst, jnp.full((pad,), _N, jnp.int32)])
    src4 = srcp.reshape(16, _NB_E // 4, 512)
    dst4 = dstp.reshape(16, _NB_E // 4, 512)
    feats_p = jnp.pad(features, ((0, _NP - _N), (0, 0)))
    h0 = _mm_call(feats_p, W, b.reshape(1, _D))
    hout, _, _ = _sc_graph(h0, src4, dst4)
    return hout[:_N]
